# baseline (speedup 1.0000x reference)
import jax
import jax.numpy as jnp
from jax.experimental import pallas as pl
from jax.experimental.pallas import tpu as pltpu

B, H, W = 4, 48, 48
HW = H * W
N = B * HW
BI = 12
RADIUS = 6.0
EPS = 1e-5


def _params_body(x_ref, mw_ref, mb_ref, hw_ref, hb_ref, bm_ref, bmt_ref,
                 col_ref, row_ref, mean_ref, det_ref, park_ref):
    x2 = x_ref[...].reshape(N, x_ref.shape[-1])
    tt = jnp.tanh(
        jnp.dot(x2, mw_ref[...], preferred_element_type=jnp.float32)
        + mb_ref[...])
    ht = jax.lax.dot_general(
        hw_ref[...], tt, (((1,), (1,)), ((), ())),
        preferred_element_type=jnp.float32) + hb_ref[...]
    xc = ht[2:4]
    inv = 1.0 / (2.0 * HW)
    m = jnp.sum(jnp.dot(xc, bmt_ref[...],
                        preferred_element_type=jnp.float32),
                axis=0, keepdims=True) * inv
    mf = jnp.dot(m, bm_ref[...],
                 preferred_element_type=jnp.float32)
    d = xc - mf
    vs = jnp.sum(jnp.dot(d * d, bmt_ref[...],
                         preferred_element_type=jnp.float32),
                 axis=0, keepdims=True) * inv
    rs = jax.lax.rsqrt(vs + EPS)
    rsf = jnp.dot(rs, bm_ref[...],
                  preferred_element_type=jnp.float32)
    s = jax.nn.sigmoid(d * rsf) * 5.0 + 0.05
    cx = s[0:1]
    cy = s[1:2]
    det = cx * cy
    det_ref[...] = det.reshape(B, HW)
    invd = (1.0 / 6.28) * jax.lax.rsqrt(det)
    mx = ht[0:1] + col_ref[...]
    my = ht[1:2] + row_ref[...]
    mean_ref[...] = jnp.concatenate([mx, my], axis=0).T.reshape(B, H, W, 2)
    park_ref[...] = jnp.concatenate(
        [mx, my, -0.5 / cx, -0.5 / cy, invd, det, det, det], axis=0).T


def _mask_body(park_ref, corr_ref, out_ref):
    pt = park_ref[...]

    def prow(k):
        return pt[:, k:k + 1].reshape(BI, W, 1)

    mx = prow(0)
    my = prow(1)
    nicx = prow(2)
    nicy = prow(3)
    ind = prow(4)
    t = jax.lax.broadcasted_iota(
        jnp.int32, (BI, W, W), 2).astype(jnp.float32)
    dx = t - mx
    dy = t - my
    a = jnp.exp(nicy * dy * dy) * (jnp.abs(dy) <= RADIUS)
    b = (jnp.exp(nicx * dx * dx) * (jnp.abs(dx) <= RADIUS)
         * ind)
    cr = corr_ref[0]
    out_ref[0] = cr + cr * (a[:, :, :, None] * b[:, :, None, :])


def kernel(x, corr, map_w, map_b, mean_w, mean_b, cov_w, cov_b):
    f32 = jnp.float32
    hw8 = jnp.concatenate(
        [mean_w, cov_w, jnp.zeros((4, mean_w.shape[1]), f32)], axis=0)
    hb8 = jnp.concatenate(
        [mean_b, cov_b, jnp.zeros((4,), f32)]).reshape(8, 1)
    bm = jnp.repeat(jnp.eye(B, dtype=f32), HW, axis=1)
    pix = jnp.arange(N, dtype=jnp.int32)
    col = (pix % W).astype(f32).reshape(1, N)
    row = ((pix // W) % H).astype(f32).reshape(1, N)

    mean, det, park2 = pl.pallas_call(
        _params_body,
        out_shape=(
            jax.ShapeDtypeStruct((B, H, W, 2), f32),
            jax.ShapeDtypeStruct((B, HW), f32),
            jax.ShapeDtypeStruct((N, 8), f32),
        ),
    )(x, map_w.T, map_b.reshape(1, -1), hw8, hb8, bm, bm.T, col, row)

    nb = H // BI
    corr1 = pl.pallas_call(
        _mask_body,
        grid=(B, nb),
        in_specs=[
            pl.BlockSpec((BI * W, 8), lambda b, i: (b * nb + i, 0)),
            pl.BlockSpec((1, BI, W, H, W), lambda b, i: (b, i, 0, 0, 0)),
        ],
        out_specs=pl.BlockSpec(
            (1, BI, W, H, W), lambda b, i: (b, i, 0, 0, 0)),
        out_shape=jax.ShapeDtypeStruct((B, H, W, H, W), f32),
        compiler_params=pltpu.CompilerParams(
            dimension_semantics=("parallel", "parallel")),
    )(park2, corr)

    return (corr1, mean, det)


# speedup vs baseline: 1.0014x; 1.0014x over previous
import jax
import jax.numpy as jnp
from jax.experimental import pallas as pl
from jax.experimental.pallas import tpu as pltpu

B, H, W = 4, 48, 48
HW = H * W
N = B * HW
BI = 8
RADIUS = 6.0
EPS = 1e-5


def _params_body(x_ref, mw_ref, mb_ref, hw_ref, hb_ref, bm_ref, bmt_ref,
                 col_ref, row_ref, mean_ref, det_ref, park_ref):
    x2 = x_ref[...].reshape(N, x_ref.shape[-1])
    tt = jnp.tanh(
        jnp.dot(x2, mw_ref[...], preferred_element_type=jnp.float32)
        + mb_ref[...])
    ht = jax.lax.dot_general(
        hw_ref[...], tt, (((1,), (1,)), ((), ())),
        preferred_element_type=jnp.float32) + hb_ref[...]
    xc = ht[2:4]
    inv = 1.0 / (2.0 * HW)
    m = jnp.sum(jnp.dot(xc, bmt_ref[...],
                        preferred_element_type=jnp.float32),
                axis=0, keepdims=True) * inv
    mf = jnp.dot(m, bm_ref[...],
                 preferred_element_type=jnp.float32)
    d = xc - mf
    vs = jnp.sum(jnp.dot(d * d, bmt_ref[...],
                         preferred_element_type=jnp.float32),
                 axis=0, keepdims=True) * inv
    rs = jax.lax.rsqrt(vs + EPS)
    rsf = jnp.dot(rs, bm_ref[...],
                  preferred_element_type=jnp.float32)
    s = jax.nn.sigmoid(d * rsf) * 5.0 + 0.05
    cx = s[0:1]
    cy = s[1:2]
    det = cx * cy
    det_ref[...] = det.reshape(B, HW)
    invd = (1.0 / 6.28) * jax.lax.rsqrt(det)
    mx = ht[0:1] + col_ref[...]
    my = ht[1:2] + row_ref[...]
    mean_ref[...] = jnp.concatenate([mx, my], axis=0).T.reshape(B, H, W, 2)
    park_ref[...] = jnp.concatenate(
        [mx, my, -0.5 / cx, -0.5 / cy, invd, det, det, det], axis=0).T


def _mask_body(park_ref, corr_ref, out_ref):
    pt = park_ref[...]

    def prow(k):
        return pt[:, k:k + 1].reshape(BI, W, 1)

    mx = prow(0)
    my = prow(1)
    nicx = prow(2)
    nicy = prow(3)
    ind = prow(4)
    t = jax.lax.broadcasted_iota(
        jnp.int32, (BI, W, W), 2).astype(jnp.float32)
    dx = t - mx
    dy = t - my
    a = jnp.exp(nicy * dy * dy) * (jnp.abs(dy) <= RADIUS)
    b = (jnp.exp(nicx * dx * dx) * (jnp.abs(dx) <= RADIUS)
         * ind)
    cr = corr_ref[0]
    out_ref[0] = cr + cr * (a[:, :, :, None] * b[:, :, None, :])


def kernel(x, corr, map_w, map_b, mean_w, mean_b, cov_w, cov_b):
    f32 = jnp.float32
    hw8 = jnp.concatenate(
        [mean_w, cov_w, jnp.zeros((4, mean_w.shape[1]), f32)], axis=0)
    hb8 = jnp.concatenate(
        [mean_b, cov_b, jnp.zeros((4,), f32)]).reshape(8, 1)
    bm = jnp.repeat(jnp.eye(B, dtype=f32), HW, axis=1)
    pix = jnp.arange(N, dtype=jnp.int32)
    col = (pix % W).astype(f32).reshape(1, N)
    row = ((pix // W) % H).astype(f32).reshape(1, N)

    mean, det, park2 = pl.pallas_call(
        _params_body,
        out_shape=(
            jax.ShapeDtypeStruct((B, H, W, 2), f32),
            jax.ShapeDtypeStruct((B, HW), f32),
            jax.ShapeDtypeStruct((N, 8), f32),
        ),
    )(x, map_w.T, map_b.reshape(1, -1), hw8, hb8, bm, bm.T, col, row)

    nb = H // BI
    corr1 = pl.pallas_call(
        _mask_body,
        grid=(B, nb),
        in_specs=[
            pl.BlockSpec((BI * W, 8), lambda b, i: (b * nb + i, 0)),
            pl.BlockSpec((1, BI, W, H, W), lambda b, i: (b, i, 0, 0, 0)),
        ],
        out_specs=pl.BlockSpec(
            (1, BI, W, H, W), lambda b, i: (b, i, 0, 0, 0)),
        out_shape=jax.ShapeDtypeStruct((B, H, W, H, W), f32),
        compiler_params=pltpu.CompilerParams(
            dimension_semantics=("parallel", "parallel")),
    )(park2, corr)

    return (corr1, mean, det)


# speedup vs baseline: 1.0568x; 1.0554x over previous
import jax
import jax.numpy as jnp
from jax.experimental import pallas as pl
from jax.experimental.pallas import tpu as pltpu

B, H, W = 4, 48, 48
HW = H * W
N = B * HW
BI = 8
NB = H // BI
RADIUS = 6.0
EPS = 1e-5


def _body(x_ref, mw_ref, mb_ref, hw_ref, hb_ref, bm_ref,
          col_ref, row_ref, corr_ref, out_ref, mean_ref, det_ref, park_s):
    first = (pl.program_id(0) == 0) & (pl.program_id(1) == 0)

    @pl.when(first)
    def _params():
        x2 = x_ref[...].reshape(N, x_ref.shape[-1])
        tt = jnp.tanh(
            jnp.dot(x2, mw_ref[...], preferred_element_type=jnp.float32)
            + mb_ref[...])
        ht = jax.lax.dot_general(
            hw_ref[...], tt, (((1,), (1,)), ((), ())),
            preferred_element_type=jnp.float32) + hb_ref[...]
        xc = ht[2:4]
        inv = 1.0 / (2.0 * HW)
        m = jnp.sum(jax.lax.dot_general(
            xc, bm_ref[...], (((1,), (1,)), ((), ())),
            preferred_element_type=jnp.float32),
                    axis=0, keepdims=True) * inv
        mf = jnp.dot(m, bm_ref[...],
                     preferred_element_type=jnp.float32)
        d = xc - mf
        vs = jnp.sum(jax.lax.dot_general(
            d * d, bm_ref[...], (((1,), (1,)), ((), ())),
            preferred_element_type=jnp.float32),
                     axis=0, keepdims=True) * inv
        rs = jax.lax.rsqrt(vs + EPS)
        rsf = jnp.dot(rs, bm_ref[...],
                      preferred_element_type=jnp.float32)
        s = jax.nn.sigmoid(d * rsf) * 5.0 + 0.05
        cx = s[0:1]
        cy = s[1:2]
        det = cx * cy
        det_ref[...] = det.reshape(B, HW)
        invd = (1.0 / 6.28) * jax.lax.rsqrt(det)
        mx = ht[0:1] + col_ref[...]
        my = ht[1:2] + row_ref[...]
        mean_ref[...] = jnp.concatenate(
            [mx, my], axis=0).T.reshape(B, H, W, 2)
        park_s[...] = jnp.concatenate(
            [mx, my, -0.5 / cx, -0.5 / cy, invd, det, det, det], axis=0).T

    g = pl.program_id(0) * NB + pl.program_id(1)
    pt = park_s[pl.ds(g * BI * W, BI * W), :]

    def prow(k):
        return pt[:, k:k + 1].reshape(BI, W, 1)

    mx = prow(0)
    my = prow(1)
    nicx = prow(2)
    nicy = prow(3)
    ind = prow(4)
    t = jax.lax.broadcasted_iota(
        jnp.int32, (BI, W, W), 2).astype(jnp.float32)
    dx = t - mx
    dy = t - my
    a = jnp.exp(nicy * dy * dy) * (jnp.abs(dy) <= RADIUS)
    b = (jnp.exp(nicx * dx * dx) * (jnp.abs(dx) <= RADIUS)
         * ind)
    cr = corr_ref[0]
    out_ref[0] = cr + cr * (a[:, :, :, None] * b[:, :, None, :])


def kernel(x, corr, map_w, map_b, mean_w, mean_b, cov_w, cov_b):
    f32 = jnp.float32
    hw8 = jnp.concatenate(
        [mean_w, cov_w, jnp.zeros((4, mean_w.shape[1]), f32)], axis=0)
    hb8 = jnp.concatenate(
        [mean_b, cov_b, jnp.zeros((4,), f32)]).reshape(8, 1)
    bm = jnp.repeat(jnp.eye(B, dtype=f32), HW, axis=1)
    pix = jnp.arange(N, dtype=jnp.int32)
    col = (pix % W).astype(f32).reshape(1, N)
    row = ((pix // W) % H).astype(f32).reshape(1, N)

    full = lambda shape: pl.BlockSpec(shape, lambda b, i: (0,) * len(shape))
    corr1, mean, det = pl.pallas_call(
        _body,
        grid=(B, NB),
        in_specs=[
            full((B, H, W, x.shape[-1])),
            full((x.shape[-1], 16)),
            full((1, 16)),
            full((8, 16)),
            full((8, 1)),
            full((B, N)),
            full((1, N)),
            full((1, N)),
            pl.BlockSpec((1, BI, W, H, W), lambda b, i: (b, i, 0, 0, 0)),
        ],
        out_specs=(
            pl.BlockSpec((1, BI, W, H, W), lambda b, i: (b, i, 0, 0, 0)),
            full((B, H, W, 2)),
            full((B, HW)),
        ),
        out_shape=(
            jax.ShapeDtypeStruct((B, H, W, H, W), f32),
            jax.ShapeDtypeStruct((B, H, W, 2), f32),
            jax.ShapeDtypeStruct((B, HW), f32),
        ),
        scratch_shapes=[pltpu.VMEM((N, 8), f32)],
        compiler_params=pltpu.CompilerParams(
            dimension_semantics=("arbitrary", "arbitrary")),
    )(x, map_w.T, map_b.reshape(1, -1), hw8, hb8, bm, col, row, corr)

    return (corr1, mean, det)
